# baseline (speedup 1.0000x reference)
import jax
import jax.numpy as jnp
import numpy as np
from jax.experimental import pallas as pl

HEADS = 4
C = 192
INNER = 3 * C
DH = INNER // HEADS
PH = PW = 8
NPP = PH * PW
NH = NW = 28
NPATCH = NH * NW
H = W = 224
HW = H * W
PYR_CELLS = 21
STRIP_TOK = NW * NPP

PIX_PER_STEP = 3584
GRID3 = HW // PIX_PER_STEP

SUB = 4
GRID1 = NH // SUB
STEP_TOK = SUB * STRIP_TOK

_F32 = jnp.float32


def _attn_kernel(x_ref, e_ref, wq_ref, wk_ref, wv_ref, wo_ref, gs_ref,
                 out_ref, sums_ref):
    for sub in range(SUB):
        sl = slice(sub * STRIP_TOK, (sub + 1) * STRIP_TOK)
        xb = x_ref[:, sl]
        t_rows = jax.lax.dot_general(e_ref[...], xb.astype(jnp.bfloat16),
                                     (((1,), (1,)), ((), ())),
                                     preferred_element_type=_F32)
        out_rows = jnp.zeros((STRIP_TOK, C), dtype=_F32)
        for h in range(HEADS):
            qh = jnp.dot(t_rows, wq_ref[h], preferred_element_type=_F32)
            kh = jnp.dot(t_rows, wk_ref[h], preferred_element_type=_F32)
            vh = jnp.dot(t_rows, wv_ref[h], preferred_element_type=_F32)
            qh = qh.reshape(NW, NPP, DH)
            kh = kh.reshape(NW, NPP, DH)
            vh = vh.reshape(NW, NPP, DH)
            s = jax.lax.dot_general(qh, kh, (((2,), (2,)), ((0,), (0,))),
                                    preferred_element_type=_F32)
            m = jnp.max(s, axis=-1, keepdims=True)
            p = jnp.exp(s - m)
            d = p / jnp.sum(p, axis=-1, keepdims=True)
            oh = jax.lax.dot_general(d, vh, (((2,), (1,)), ((0,), (0,))),
                                     preferred_element_type=_F32)
            out_rows = out_rows + jnp.dot(oh.reshape(STRIP_TOK, DH),
                                          wo_ref[h],
                                          preferred_element_type=_F32)
        out_ref[:, sl] = jax.lax.dot_general(out_rows.astype(jnp.bfloat16),
                                             e_ref[...],
                                             (((0,), (0,)), ((), ())),
                                             preferred_element_type=_F32)
        sums_ref[:, 0, sub, :] = jnp.dot(xb, gs_ref[...],
                                         preferred_element_type=_F32)


def _ctx_kernel(sums_ref, m_ref, wf_ref, fc_ref):
    sums = sums_ref[...].reshape(C, NPATCH)
    pooled = jnp.dot(sums, m_ref[...], preferred_element_type=_F32)
    feats = jax.lax.dot_general(wf_ref[...], pooled,
                                (((0,), (0,)), ((), ())),
                                preferred_element_type=_F32)
    mx = jnp.max(feats, axis=-1, keepdims=True)
    e = jnp.exp(feats - mx)
    keys = e / jnp.sum(e, axis=-1, keepdims=True)
    fc_ref[...] = jax.lax.dot_general(feats, keys, (((1,), (1,)), ((), ())),
                                      preferred_element_type=_F32)


def _fmam_kernel(x_ref, spa_ref, fc_ref, wq_ref, bq_ref, wdw_ref, out_ref):
    x = x_ref[...]
    qf = jax.lax.dot_general(wq_ref[...], x, (((0,), (0,)), ((), ())),
                             preferred_element_type=_F32)
    qf = qf + bq_ref[...]
    mx = jnp.max(qf, axis=0, keepdims=True)
    e = jnp.exp(qf - mx)
    qf = e / jnp.sum(e, axis=0, keepdims=True)
    fa = jax.lax.dot_general(fc_ref[...], qf, (((0,), (0,)), ((), ())),
                             preferred_element_type=_F32)
    w0 = wdw_ref[:, 0:1]
    w1 = wdw_ref[:, 1:2]
    out_ref[...] = spa_ref[...] * w0 + fa * w1


def _perm_matrix():
    e = np.zeros((STRIP_TOK, STRIP_TOK), dtype=np.float32)
    for hh in range(PH):
        for pw_ in range(NW):
            for ww in range(PW):
                n = pw_ * NPP + hh * PW + ww
                l = hh * W + pw_ * PW + ww
                e[n, l] = 1.0
    return e


def _strip_sum_matrix():
    g = np.zeros((STRIP_TOK, NW), dtype=np.float32)
    for l in range(STRIP_TOK):
        g[l, (l % W) // PW] = 1.0
    return g


def _pool_matrix():
    m = np.zeros((NPATCH, PYR_CELLS), dtype=np.float32)
    col = 0
    for lvl in range(3):
        s = 2 ** lvl
        pps = NH // s
        npx = (H // s) * (W // s)
        for i in range(s):
            for j in range(s):
                for ph in range(i * pps, (i + 1) * pps):
                    for pw_ in range(j * pps, (j + 1) * pps):
                        m[ph * NW + pw_, col] = 1.0 / npx
                col += 1
    return m


def kernel(x, Wq, Wk, Wv, Wo, Wquer, bquer, Wf, Wdw, alpha, beta):
    del alpha, beta
    x2d = x.reshape(C, HW)
    Wq4 = Wq.reshape(C, HEADS, DH).transpose(1, 0, 2)
    Wk4 = Wk.reshape(C, HEADS, DH).transpose(1, 0, 2)
    Wv4 = Wv.reshape(C, HEADS, DH).transpose(1, 0, 2)
    Wo4 = Wo.reshape(HEADS, DH, C)

    spa2d, sums = pl.pallas_call(
        _attn_kernel,
        grid=(GRID1,),
        in_specs=[
            pl.BlockSpec((C, STEP_TOK), lambda i: (0, i)),
            pl.BlockSpec((STRIP_TOK, STRIP_TOK), lambda i: (0, 0)),
            pl.BlockSpec((HEADS, C, DH), lambda i: (0, 0, 0)),
            pl.BlockSpec((HEADS, C, DH), lambda i: (0, 0, 0)),
            pl.BlockSpec((HEADS, C, DH), lambda i: (0, 0, 0)),
            pl.BlockSpec((HEADS, DH, C), lambda i: (0, 0, 0)),
            pl.BlockSpec((STRIP_TOK, NW), lambda i: (0, 0)),
        ],
        out_specs=[
            pl.BlockSpec((C, STEP_TOK), lambda i: (0, i)),
            pl.BlockSpec((C, 1, SUB, NW), lambda i: (0, i, 0, 0)),
        ],
        out_shape=[
            jax.ShapeDtypeStruct((C, HW), _F32),
            jax.ShapeDtypeStruct((C, GRID1, SUB, NW), _F32),
        ],
    )(x2d, jnp.asarray(_perm_matrix(), dtype=jnp.bfloat16), Wq4, Wk4, Wv4,
      Wo4, jnp.asarray(_strip_sum_matrix()))

    pool_m = jnp.asarray(_pool_matrix())
    fc = pl.pallas_call(
        _ctx_kernel,
        out_shape=jax.ShapeDtypeStruct((C, C), _F32),
    )(sums, pool_m, Wf)

    out = pl.pallas_call(
        _fmam_kernel,
        grid=(GRID3,),
        in_specs=[
            pl.BlockSpec((C, PIX_PER_STEP), lambda i: (0, i)),
            pl.BlockSpec((C, PIX_PER_STEP), lambda i: (0, i)),
            pl.BlockSpec((C, C), lambda i: (0, 0)),
            pl.BlockSpec((C, C), lambda i: (0, 0)),
            pl.BlockSpec((C, 1), lambda i: (0, 0)),
            pl.BlockSpec((C, 2), lambda i: (0, 0)),
        ],
        out_specs=pl.BlockSpec((C, PIX_PER_STEP), lambda i: (0, i)),
        out_shape=jax.ShapeDtypeStruct((C, HW), _F32),
    )(x2d, spa2d, fc, Wquer, bquer.reshape(C, 1), Wdw)

    return out.reshape(1, C, H, W)


# speedup vs baseline: 1.5638x; 1.5638x over previous
import jax
import jax.numpy as jnp
import numpy as np
from jax.experimental import pallas as pl

HEADS = 4
C = 192
INNER = 3 * C
DH = INNER // HEADS
PH = PW = 8
NPP = PH * PW
NH = NW = 28
NPATCH = NH * NW
H = W = 224
HW = H * W
PYR_CELLS = 21
STRIP_TOK = NW * NPP

PIX_PER_STEP = 3584
GRID3 = HW // PIX_PER_STEP

SUB = 1
GRID1 = NH // SUB
STEP_TOK = SUB * STRIP_TOK

_F32 = jnp.float32


def _attn_kernel(x_ref, i_ref, wq_ref, wk_ref, wv_ref, wo_ref, gs_ref,
                 out_ref, sums_ref):
    for sub in range(SUB):
        sl = slice(sub * STRIP_TOK, (sub + 1) * STRIP_TOK)
        xb = x_ref[:, sl]
        rows = jax.lax.dot_general(xb, i_ref[...], (((0,), (0,)), ((), ())),
                                   preferred_element_type=_F32)
        t_rows = (rows.reshape(PH, NW, PW, C)
                      .transpose(1, 0, 2, 3)
                      .reshape(STRIP_TOK, C))
        out_rows = jnp.zeros((STRIP_TOK, C), dtype=_F32)
        for h in range(HEADS):
            qh = jnp.dot(t_rows, wq_ref[h], preferred_element_type=_F32)
            kh = jnp.dot(t_rows, wk_ref[h], preferred_element_type=_F32)
            vh = jnp.dot(t_rows, wv_ref[h], preferred_element_type=_F32)
            qh = qh.reshape(NW, NPP, DH)
            kh = kh.reshape(NW, NPP, DH)
            vh = vh.reshape(NW, NPP, DH)
            s = jax.lax.dot_general(qh, kh, (((2,), (2,)), ((0,), (0,))),
                                    preferred_element_type=_F32)
            m = jnp.max(s, axis=-1, keepdims=True)
            p = jnp.exp(s - m)
            d = p / jnp.sum(p, axis=-1, keepdims=True)
            oh = jax.lax.dot_general(d, vh, (((2,), (1,)), ((0,), (0,))),
                                     preferred_element_type=_F32)
            out_rows = out_rows + jnp.dot(oh.reshape(STRIP_TOK, DH),
                                          wo_ref[h],
                                          preferred_element_type=_F32)
        back = (out_rows.reshape(NW, PH, PW, C)
                        .transpose(1, 0, 2, 3)
                        .reshape(STRIP_TOK, C))
        out_ref[:, sl] = jax.lax.dot_general(i_ref[...], back,
                                             (((1,), (1,)), ((), ())),
                                             preferred_element_type=_F32)
        sums_ref[:, 0, sub, :] = jnp.dot(xb, gs_ref[...],
                                         preferred_element_type=_F32)


def _ctx_kernel(sums_ref, m_ref, wf_ref, fc_ref):
    sums = sums_ref[...].reshape(C, NPATCH)
    pooled = jnp.dot(sums, m_ref[...], preferred_element_type=_F32)
    feats = jax.lax.dot_general(wf_ref[...], pooled,
                                (((0,), (0,)), ((), ())),
                                preferred_element_type=_F32)
    mx = jnp.max(feats, axis=-1, keepdims=True)
    e = jnp.exp(feats - mx)
    keys = e / jnp.sum(e, axis=-1, keepdims=True)
    fc_ref[...] = jax.lax.dot_general(feats, keys, (((1,), (1,)), ((), ())),
                                      preferred_element_type=_F32)


def _fmam_kernel(x_ref, spa_ref, fc_ref, wq_ref, bq_ref, wdw_ref, out_ref):
    x = x_ref[...]
    qf = jax.lax.dot_general(wq_ref[...], x, (((0,), (0,)), ((), ())),
                             preferred_element_type=_F32)
    qf = qf + bq_ref[...]
    mx = jnp.max(qf, axis=0, keepdims=True)
    e = jnp.exp(qf - mx)
    qf = e / jnp.sum(e, axis=0, keepdims=True)
    fa = jax.lax.dot_general(fc_ref[...], qf, (((0,), (0,)), ((), ())),
                             preferred_element_type=_F32)
    w0 = wdw_ref[:, 0:1]
    w1 = wdw_ref[:, 1:2]
    out_ref[...] = spa_ref[...] * w0 + fa * w1


def _perm_matrix():
    e = np.zeros((STRIP_TOK, STRIP_TOK), dtype=np.float32)
    for hh in range(PH):
        for pw_ in range(NW):
            for ww in range(PW):
                n = pw_ * NPP + hh * PW + ww
                l = hh * W + pw_ * PW + ww
                e[n, l] = 1.0
    return e


def _strip_sum_matrix():
    g = np.zeros((STRIP_TOK, NW), dtype=np.float32)
    for l in range(STRIP_TOK):
        g[l, (l % W) // PW] = 1.0
    return g


def _pool_matrix():
    m = np.zeros((NPATCH, PYR_CELLS), dtype=np.float32)
    col = 0
    for lvl in range(3):
        s = 2 ** lvl
        pps = NH // s
        npx = (H // s) * (W // s)
        for i in range(s):
            for j in range(s):
                for ph in range(i * pps, (i + 1) * pps):
                    for pw_ in range(j * pps, (j + 1) * pps):
                        m[ph * NW + pw_, col] = 1.0 / npx
                col += 1
    return m


def kernel(x, Wq, Wk, Wv, Wo, Wquer, bquer, Wf, Wdw, alpha, beta):
    del alpha, beta
    x2d = x.reshape(C, HW)
    Wq4 = Wq.reshape(C, HEADS, DH).transpose(1, 0, 2)
    Wk4 = Wk.reshape(C, HEADS, DH).transpose(1, 0, 2)
    Wv4 = Wv.reshape(C, HEADS, DH).transpose(1, 0, 2)
    Wo4 = Wo.reshape(HEADS, DH, C)

    spa2d, sums = pl.pallas_call(
        _attn_kernel,
        grid=(GRID1,),
        in_specs=[
            pl.BlockSpec((C, STEP_TOK), lambda i: (0, i)),
            pl.BlockSpec((C, C), lambda i: (0, 0)),
            pl.BlockSpec((HEADS, C, DH), lambda i: (0, 0, 0)),
            pl.BlockSpec((HEADS, C, DH), lambda i: (0, 0, 0)),
            pl.BlockSpec((HEADS, C, DH), lambda i: (0, 0, 0)),
            pl.BlockSpec((HEADS, DH, C), lambda i: (0, 0, 0)),
            pl.BlockSpec((STRIP_TOK, NW), lambda i: (0, 0)),
        ],
        out_specs=[
            pl.BlockSpec((C, STEP_TOK), lambda i: (0, i)),
            pl.BlockSpec((C, 1, SUB, NW), lambda i: (0, i, 0, 0)),
        ],
        out_shape=[
            jax.ShapeDtypeStruct((C, HW), _F32),
            jax.ShapeDtypeStruct((C, GRID1, SUB, NW), _F32),
        ],
    )(x2d, jnp.eye(C, dtype=_F32), Wq4, Wk4, Wv4,
      Wo4, jnp.asarray(_strip_sum_matrix()))

    pool_m = jnp.asarray(_pool_matrix())
    fc = pl.pallas_call(
        _ctx_kernel,
        out_shape=jax.ShapeDtypeStruct((C, C), _F32),
    )(sums, pool_m, Wf)

    out = pl.pallas_call(
        _fmam_kernel,
        grid=(GRID3,),
        in_specs=[
            pl.BlockSpec((C, PIX_PER_STEP), lambda i: (0, i)),
            pl.BlockSpec((C, PIX_PER_STEP), lambda i: (0, i)),
            pl.BlockSpec((C, C), lambda i: (0, 0)),
            pl.BlockSpec((C, C), lambda i: (0, 0)),
            pl.BlockSpec((C, 1), lambda i: (0, 0)),
            pl.BlockSpec((C, 2), lambda i: (0, 0)),
        ],
        out_specs=pl.BlockSpec((C, PIX_PER_STEP), lambda i: (0, i)),
        out_shape=jax.ShapeDtypeStruct((C, HW), _F32),
    )(x2d, spa2d, fc, Wquer, bquer.reshape(C, 1), Wdw)

    return out.reshape(1, C, H, W)


# speedup vs baseline: 1.5817x; 1.0114x over previous
import jax
import jax.numpy as jnp
import numpy as np
from jax.experimental import pallas as pl

HEADS = 4
C = 192
INNER = 3 * C
DH = INNER // HEADS
PH = PW = 8
NPP = PH * PW
NH = NW = 28
NPATCH = NH * NW
H = W = 224
HW = H * W
PYR_CELLS = 21
STRIP_TOK = NW * NPP

PIX_PER_STEP = 3584
GRID3 = HW // PIX_PER_STEP

SUB = 1
GRID1 = NH // SUB
STEP_TOK = SUB * STRIP_TOK

_F32 = jnp.float32


def _attn_kernel(x_ref, i_ref, wq_ref, wk_ref, wv_ref, wo_ref, gs_ref,
                 out_ref, sums_ref):
    for sub in range(SUB):
        sl = slice(sub * STRIP_TOK, (sub + 1) * STRIP_TOK)
        xb = x_ref[:, sl]
        rows = jax.lax.dot_general(xb, i_ref[...], (((0,), (0,)), ((), ())),
                                   preferred_element_type=_F32)
        t_rows = (rows.reshape(PH, NW, PW, C)
                      .transpose(1, 0, 2, 3)
                      .reshape(STRIP_TOK, C))
        out_rows = jnp.zeros((STRIP_TOK, C), dtype=_F32)
        for h in range(HEADS):
            qh = jnp.dot(t_rows, wq_ref[h], preferred_element_type=_F32)
            kh = jnp.dot(t_rows, wk_ref[h], preferred_element_type=_F32)
            vh = jnp.dot(t_rows, wv_ref[h], preferred_element_type=_F32)
            qh = qh.reshape(NW, NPP, DH)
            kh = kh.reshape(NW, NPP, DH)
            vh = vh.reshape(NW, NPP, DH)
            s = jax.lax.dot_general(qh, kh, (((2,), (2,)), ((0,), (0,))),
                                    preferred_element_type=_F32)
            m = jnp.max(s, axis=-1, keepdims=True)
            p = jnp.exp(s - m)
            d = p / jnp.sum(p, axis=-1, keepdims=True)
            oh = jax.lax.dot_general(d, vh, (((2,), (1,)), ((0,), (0,))),
                                     preferred_element_type=_F32)
            out_rows = out_rows + jnp.dot(oh.reshape(STRIP_TOK, DH),
                                          wo_ref[h],
                                          preferred_element_type=_F32)
        back = (out_rows.reshape(NW, PH, PW, C)
                        .transpose(1, 0, 2, 3)
                        .reshape(STRIP_TOK, C))
        out_ref[:, sl] = jax.lax.dot_general(
            i_ref[...], back, (((1,), (1,)), ((), ())),
            preferred_element_type=_F32).astype(jnp.bfloat16)
        sums_ref[:, 0, sub, :] = jnp.dot(xb, gs_ref[...],
                                         preferred_element_type=_F32)


def _ctx_kernel(sums_ref, m_ref, wf_ref, fc_ref):
    sums = sums_ref[...].reshape(C, NPATCH)
    pooled = jnp.dot(sums, m_ref[...], preferred_element_type=_F32)
    feats = jax.lax.dot_general(wf_ref[...], pooled,
                                (((0,), (0,)), ((), ())),
                                preferred_element_type=_F32)
    mx = jnp.max(feats, axis=-1, keepdims=True)
    e = jnp.exp(feats - mx)
    keys = e / jnp.sum(e, axis=-1, keepdims=True)
    fc_ref[...] = jax.lax.dot_general(feats, keys, (((1,), (1,)), ((), ())),
                                      preferred_element_type=_F32)


def _fmam_kernel(x_ref, spa_ref, fc_ref, wq_ref, bq_ref, wdw_ref, out_ref):
    x = x_ref[...]
    qf = jax.lax.dot_general(wq_ref[...], x, (((0,), (0,)), ((), ())),
                             preferred_element_type=_F32)
    qf = qf + bq_ref[...]
    mx = jnp.max(qf, axis=0, keepdims=True)
    e = jnp.exp(qf - mx)
    qf = e / jnp.sum(e, axis=0, keepdims=True)
    fa = jax.lax.dot_general(fc_ref[...], qf, (((0,), (0,)), ((), ())),
                             preferred_element_type=_F32)
    w0 = wdw_ref[:, 0:1]
    w1 = wdw_ref[:, 1:2]
    out_ref[...] = spa_ref[...].astype(_F32) * w0 + fa * w1


def _perm_matrix():
    e = np.zeros((STRIP_TOK, STRIP_TOK), dtype=np.float32)
    for hh in range(PH):
        for pw_ in range(NW):
            for ww in range(PW):
                n = pw_ * NPP + hh * PW + ww
                l = hh * W + pw_ * PW + ww
                e[n, l] = 1.0
    return e


def _strip_sum_matrix():
    g = np.zeros((STRIP_TOK, NW), dtype=np.float32)
    for l in range(STRIP_TOK):
        g[l, (l % W) // PW] = 1.0
    return g


def _pool_matrix():
    m = np.zeros((NPATCH, PYR_CELLS), dtype=np.float32)
    col = 0
    for lvl in range(3):
        s = 2 ** lvl
        pps = NH // s
        npx = (H // s) * (W // s)
        for i in range(s):
            for j in range(s):
                for ph in range(i * pps, (i + 1) * pps):
                    for pw_ in range(j * pps, (j + 1) * pps):
                        m[ph * NW + pw_, col] = 1.0 / npx
                col += 1
    return m


def kernel(x, Wq, Wk, Wv, Wo, Wquer, bquer, Wf, Wdw, alpha, beta):
    del alpha, beta
    x2d = x.reshape(C, HW)
    Wq4 = Wq.reshape(C, HEADS, DH).transpose(1, 0, 2)
    Wk4 = Wk.reshape(C, HEADS, DH).transpose(1, 0, 2)
    Wv4 = Wv.reshape(C, HEADS, DH).transpose(1, 0, 2)
    Wo4 = Wo.reshape(HEADS, DH, C)

    spa2d, sums = pl.pallas_call(
        _attn_kernel,
        grid=(GRID1,),
        in_specs=[
            pl.BlockSpec((C, STEP_TOK), lambda i: (0, i)),
            pl.BlockSpec((C, C), lambda i: (0, 0)),
            pl.BlockSpec((HEADS, C, DH), lambda i: (0, 0, 0)),
            pl.BlockSpec((HEADS, C, DH), lambda i: (0, 0, 0)),
            pl.BlockSpec((HEADS, C, DH), lambda i: (0, 0, 0)),
            pl.BlockSpec((HEADS, DH, C), lambda i: (0, 0, 0)),
            pl.BlockSpec((STRIP_TOK, NW), lambda i: (0, 0)),
        ],
        out_specs=[
            pl.BlockSpec((C, STEP_TOK), lambda i: (0, i)),
            pl.BlockSpec((C, 1, SUB, NW), lambda i: (0, i, 0, 0)),
        ],
        out_shape=[
            jax.ShapeDtypeStruct((C, HW), jnp.bfloat16),
            jax.ShapeDtypeStruct((C, GRID1, SUB, NW), _F32),
        ],
    )(x2d, jnp.eye(C, dtype=_F32), Wq4, Wk4, Wv4,
      Wo4, jnp.asarray(_strip_sum_matrix()))

    pool_m = jnp.asarray(_pool_matrix())
    fc = pl.pallas_call(
        _ctx_kernel,
        out_shape=jax.ShapeDtypeStruct((C, C), _F32),
    )(sums, pool_m, Wf)

    out = pl.pallas_call(
        _fmam_kernel,
        grid=(GRID3,),
        in_specs=[
            pl.BlockSpec((C, PIX_PER_STEP), lambda i: (0, i)),
            pl.BlockSpec((C, PIX_PER_STEP), lambda i: (0, i)),
            pl.BlockSpec((C, C), lambda i: (0, 0)),
            pl.BlockSpec((C, C), lambda i: (0, 0)),
            pl.BlockSpec((C, 1), lambda i: (0, 0)),
            pl.BlockSpec((C, 2), lambda i: (0, 0)),
        ],
        out_specs=pl.BlockSpec((C, PIX_PER_STEP), lambda i: (0, i)),
        out_shape=jax.ShapeDtypeStruct((C, HW), _F32),
    )(x2d, spa2d, fc, Wquer, bquer.reshape(C, 1), Wdw)

    return out.reshape(1, C, H, W)


# speedup vs baseline: 1.5935x; 1.0075x over previous
import jax
import jax.numpy as jnp
import numpy as np
from jax.experimental import pallas as pl

HEADS = 4
C = 192
INNER = 3 * C
DH = INNER // HEADS
PH = PW = 8
NPP = PH * PW
NH = NW = 28
NPATCH = NH * NW
H = W = 224
HW = H * W
PYR_CELLS = 21
STRIP_TOK = NW * NPP

PIX_PER_STEP = 3584
GRID3 = HW // PIX_PER_STEP

SUB = 2
GRID1 = NH // SUB
STEP_TOK = SUB * STRIP_TOK

_F32 = jnp.float32


def _attn_kernel(x_ref, i_ref, wq_ref, wk_ref, wv_ref, wo_ref, gs_ref,
                 out_ref, sums_ref):
    for sub in range(SUB):
        sl = slice(sub * STRIP_TOK, (sub + 1) * STRIP_TOK)
        xb = x_ref[:, sl]
        rows = jax.lax.dot_general(xb, i_ref[...], (((0,), (0,)), ((), ())),
                                   preferred_element_type=_F32)
        t_rows = (rows.reshape(PH, NW, PW, C)
                      .transpose(1, 0, 2, 3)
                      .reshape(STRIP_TOK, C))
        out_rows = jnp.zeros((STRIP_TOK, C), dtype=_F32)
        for h in range(HEADS):
            qh = jnp.dot(t_rows, wq_ref[h], preferred_element_type=_F32)
            kh = jnp.dot(t_rows, wk_ref[h], preferred_element_type=_F32)
            vh = jnp.dot(t_rows, wv_ref[h], preferred_element_type=_F32)
            qh = qh.reshape(NW, NPP, DH)
            kh = kh.reshape(NW, NPP, DH)
            vh = vh.reshape(NW, NPP, DH)
            s = jax.lax.dot_general(qh, kh, (((2,), (2,)), ((0,), (0,))),
                                    preferred_element_type=_F32)
            m = jnp.max(s, axis=-1, keepdims=True)
            p = jnp.exp(s - m)
            d = p / jnp.sum(p, axis=-1, keepdims=True)
            oh = jax.lax.dot_general(d, vh, (((2,), (1,)), ((0,), (0,))),
                                     preferred_element_type=_F32)
            out_rows = out_rows + jnp.dot(oh.reshape(STRIP_TOK, DH),
                                          wo_ref[h],
                                          preferred_element_type=_F32)
        back = (out_rows.reshape(NW, PH, PW, C)
                        .transpose(1, 0, 2, 3)
                        .reshape(STRIP_TOK, C))
        out_ref[:, sl] = jax.lax.dot_general(
            i_ref[...], back, (((1,), (1,)), ((), ())),
            preferred_element_type=_F32).astype(jnp.bfloat16)
        sums_ref[:, 0, sub, :] = jnp.dot(xb, gs_ref[...],
                                         preferred_element_type=_F32)


def _ctx_kernel(sums_ref, m_ref, wf_ref, fc_ref):
    sums = sums_ref[...].reshape(C, NPATCH)
    pooled = jnp.dot(sums, m_ref[...], preferred_element_type=_F32)
    feats = jax.lax.dot_general(wf_ref[...], pooled,
                                (((0,), (0,)), ((), ())),
                                preferred_element_type=_F32)
    mx = jnp.max(feats, axis=-1, keepdims=True)
    e = jnp.exp(feats - mx)
    keys = e / jnp.sum(e, axis=-1, keepdims=True)
    fc_ref[...] = jax.lax.dot_general(feats, keys, (((1,), (1,)), ((), ())),
                                      preferred_element_type=_F32)


def _fmam_kernel(x_ref, spa_ref, fc_ref, wq_ref, bq_ref, wdw_ref, out_ref):
    x = x_ref[...]
    qf = jax.lax.dot_general(wq_ref[...], x, (((0,), (0,)), ((), ())),
                             preferred_element_type=_F32)
    qf = qf + bq_ref[...]
    mx = jnp.max(qf, axis=0, keepdims=True)
    e = jnp.exp(qf - mx)
    qf = e / jnp.sum(e, axis=0, keepdims=True)
    fa = jax.lax.dot_general(fc_ref[...], qf, (((0,), (0,)), ((), ())),
                             preferred_element_type=_F32)
    w0 = wdw_ref[:, 0:1]
    w1 = wdw_ref[:, 1:2]
    out_ref[...] = spa_ref[...].astype(_F32) * w0 + fa * w1


def _perm_matrix():
    e = np.zeros((STRIP_TOK, STRIP_TOK), dtype=np.float32)
    for hh in range(PH):
        for pw_ in range(NW):
            for ww in range(PW):
                n = pw_ * NPP + hh * PW + ww
                l = hh * W + pw_ * PW + ww
                e[n, l] = 1.0
    return e


def _strip_sum_matrix():
    g = np.zeros((STRIP_TOK, NW), dtype=np.float32)
    for l in range(STRIP_TOK):
        g[l, (l % W) // PW] = 1.0
    return g


def _pool_matrix():
    m = np.zeros((NPATCH, PYR_CELLS), dtype=np.float32)
    col = 0
    for lvl in range(3):
        s = 2 ** lvl
        pps = NH // s
        npx = (H // s) * (W // s)
        for i in range(s):
            for j in range(s):
                for ph in range(i * pps, (i + 1) * pps):
                    for pw_ in range(j * pps, (j + 1) * pps):
                        m[ph * NW + pw_, col] = 1.0 / npx
                col += 1
    return m


def kernel(x, Wq, Wk, Wv, Wo, Wquer, bquer, Wf, Wdw, alpha, beta):
    del alpha, beta
    x2d = x.reshape(C, HW)
    Wq4 = Wq.reshape(C, HEADS, DH).transpose(1, 0, 2)
    Wk4 = Wk.reshape(C, HEADS, DH).transpose(1, 0, 2)
    Wv4 = Wv.reshape(C, HEADS, DH).transpose(1, 0, 2)
    Wo4 = Wo.reshape(HEADS, DH, C)

    spa2d, sums = pl.pallas_call(
        _attn_kernel,
        grid=(GRID1,),
        in_specs=[
            pl.BlockSpec((C, STEP_TOK), lambda i: (0, i)),
            pl.BlockSpec((C, C), lambda i: (0, 0)),
            pl.BlockSpec((HEADS, C, DH), lambda i: (0, 0, 0)),
            pl.BlockSpec((HEADS, C, DH), lambda i: (0, 0, 0)),
            pl.BlockSpec((HEADS, C, DH), lambda i: (0, 0, 0)),
            pl.BlockSpec((HEADS, DH, C), lambda i: (0, 0, 0)),
            pl.BlockSpec((STRIP_TOK, NW), lambda i: (0, 0)),
        ],
        out_specs=[
            pl.BlockSpec((C, STEP_TOK), lambda i: (0, i)),
            pl.BlockSpec((C, 1, SUB, NW), lambda i: (0, i, 0, 0)),
        ],
        out_shape=[
            jax.ShapeDtypeStruct((C, HW), jnp.bfloat16),
            jax.ShapeDtypeStruct((C, GRID1, SUB, NW), _F32),
        ],
    )(x2d, jnp.eye(C, dtype=_F32), Wq4, Wk4, Wv4,
      Wo4, jnp.asarray(_strip_sum_matrix()))

    pool_m = jnp.asarray(_pool_matrix())
    fc = pl.pallas_call(
        _ctx_kernel,
        out_shape=jax.ShapeDtypeStruct((C, C), _F32),
    )(sums, pool_m, Wf)

    out = pl.pallas_call(
        _fmam_kernel,
        grid=(GRID3,),
        in_specs=[
            pl.BlockSpec((C, PIX_PER_STEP), lambda i: (0, i)),
            pl.BlockSpec((C, PIX_PER_STEP), lambda i: (0, i)),
            pl.BlockSpec((C, C), lambda i: (0, 0)),
            pl.BlockSpec((C, C), lambda i: (0, 0)),
            pl.BlockSpec((C, 1), lambda i: (0, 0)),
            pl.BlockSpec((C, 2), lambda i: (0, 0)),
        ],
        out_specs=pl.BlockSpec((C, PIX_PER_STEP), lambda i: (0, i)),
        out_shape=jax.ShapeDtypeStruct((C, HW), _F32),
    )(x2d, spa2d, fc, Wquer, bquer.reshape(C, 1), Wdw)

    return out.reshape(1, C, H, W)


# speedup vs baseline: 1.5957x; 1.0014x over previous
import jax
import jax.numpy as jnp
import numpy as np
from jax.experimental import pallas as pl

HEADS = 4
C = 192
INNER = 3 * C
DH = INNER // HEADS
PH = PW = 8
NPP = PH * PW
NH = NW = 28
NPATCH = NH * NW
H = W = 224
HW = H * W
PYR_CELLS = 21
STRIP_TOK = NW * NPP

PIX_PER_STEP = 7168
GRID3 = HW // PIX_PER_STEP

SUB = 2
GRID1 = NH // SUB
STEP_TOK = SUB * STRIP_TOK

_F32 = jnp.float32


def _attn_kernel(x_ref, i_ref, wq_ref, wk_ref, wv_ref, wo_ref, gs_ref,
                 out_ref, sums_ref):
    for sub in range(SUB):
        sl = slice(sub * STRIP_TOK, (sub + 1) * STRIP_TOK)
        xb = x_ref[:, sl]
        rows = jax.lax.dot_general(xb, i_ref[...], (((0,), (0,)), ((), ())),
                                   preferred_element_type=_F32)
        t_rows = (rows.reshape(PH, NW, PW, C)
                      .transpose(1, 0, 2, 3)
                      .reshape(STRIP_TOK, C))
        out_rows = jnp.zeros((STRIP_TOK, C), dtype=_F32)
        for h in range(HEADS):
            qh = jnp.dot(t_rows, wq_ref[h], preferred_element_type=_F32)
            kh = jnp.dot(t_rows, wk_ref[h], preferred_element_type=_F32)
            vh = jnp.dot(t_rows, wv_ref[h], preferred_element_type=_F32)
            qh = qh.reshape(NW, NPP, DH)
            kh = kh.reshape(NW, NPP, DH)
            vh = vh.reshape(NW, NPP, DH)
            s = jax.lax.dot_general(qh, kh, (((2,), (2,)), ((0,), (0,))),
                                    preferred_element_type=_F32)
            m = jnp.max(s, axis=-1, keepdims=True)
            p = jnp.exp(s - m)
            d = p / jnp.sum(p, axis=-1, keepdims=True)
            oh = jax.lax.dot_general(d, vh, (((2,), (1,)), ((0,), (0,))),
                                     preferred_element_type=_F32)
            out_rows = out_rows + jnp.dot(oh.reshape(STRIP_TOK, DH),
                                          wo_ref[h],
                                          preferred_element_type=_F32)
        back = (out_rows.reshape(NW, PH, PW, C)
                        .transpose(1, 0, 2, 3)
                        .reshape(STRIP_TOK, C))
        out_ref[:, sl] = jax.lax.dot_general(
            i_ref[...], back, (((1,), (1,)), ((), ())),
            preferred_element_type=_F32).astype(jnp.bfloat16)
        sums_ref[:, 0, sub, :] = jnp.dot(xb, gs_ref[...],
                                         preferred_element_type=_F32)


def _ctx_kernel(sums_ref, m_ref, wf_ref, fc_ref):
    sums = sums_ref[...].reshape(C, NPATCH)
    pooled = jnp.dot(sums, m_ref[...], preferred_element_type=_F32)
    feats = jax.lax.dot_general(wf_ref[...], pooled,
                                (((0,), (0,)), ((), ())),
                                preferred_element_type=_F32)
    mx = jnp.max(feats, axis=-1, keepdims=True)
    e = jnp.exp(feats - mx)
    keys = e / jnp.sum(e, axis=-1, keepdims=True)
    fc_ref[...] = jax.lax.dot_general(feats, keys, (((1,), (1,)), ((), ())),
                                      preferred_element_type=_F32)


def _fmam_kernel(x_ref, spa_ref, fc_ref, wq_ref, bq_ref, wdw_ref, out_ref):
    x = x_ref[...]
    qf = jax.lax.dot_general(wq_ref[...], x, (((0,), (0,)), ((), ())),
                             preferred_element_type=_F32)
    qf = qf + bq_ref[...]
    mx = jnp.max(qf, axis=0, keepdims=True)
    e = jnp.exp(qf - mx)
    qf = e / jnp.sum(e, axis=0, keepdims=True)
    fa = jax.lax.dot_general(fc_ref[...], qf, (((0,), (0,)), ((), ())),
                             preferred_element_type=_F32)
    w0 = wdw_ref[:, 0:1]
    w1 = wdw_ref[:, 1:2]
    out_ref[...] = spa_ref[...].astype(_F32) * w0 + fa * w1


def _strip_sum_matrix():
    g = np.zeros((STRIP_TOK, NW), dtype=np.float32)
    for l in range(STRIP_TOK):
        g[l, (l % W) // PW] = 1.0
    return g


def _pool_matrix():
    m = np.zeros((NPATCH, PYR_CELLS), dtype=np.float32)
    col = 0
    for lvl in range(3):
        s = 2 ** lvl
        pps = NH // s
        npx = (H // s) * (W // s)
        for i in range(s):
            for j in range(s):
                for ph in range(i * pps, (i + 1) * pps):
                    for pw_ in range(j * pps, (j + 1) * pps):
                        m[ph * NW + pw_, col] = 1.0 / npx
                col += 1
    return m


def kernel(x, Wq, Wk, Wv, Wo, Wquer, bquer, Wf, Wdw, alpha, beta):
    del alpha, beta
    x2d = x.reshape(C, HW)
    Wq4 = Wq.reshape(C, HEADS, DH).transpose(1, 0, 2)
    Wk4 = Wk.reshape(C, HEADS, DH).transpose(1, 0, 2)
    Wv4 = Wv.reshape(C, HEADS, DH).transpose(1, 0, 2)
    Wo4 = Wo.reshape(HEADS, DH, C)

    spa2d, sums = pl.pallas_call(
        _attn_kernel,
        grid=(GRID1,),
        in_specs=[
            pl.BlockSpec((C, STEP_TOK), lambda i: (0, i)),
            pl.BlockSpec((C, C), lambda i: (0, 0)),
            pl.BlockSpec((HEADS, C, DH), lambda i: (0, 0, 0)),
            pl.BlockSpec((HEADS, C, DH), lambda i: (0, 0, 0)),
            pl.BlockSpec((HEADS, C, DH), lambda i: (0, 0, 0)),
            pl.BlockSpec((HEADS, DH, C), lambda i: (0, 0, 0)),
            pl.BlockSpec((STRIP_TOK, NW), lambda i: (0, 0)),
        ],
        out_specs=[
            pl.BlockSpec((C, STEP_TOK), lambda i: (0, i)),
            pl.BlockSpec((C, 1, SUB, NW), lambda i: (0, i, 0, 0)),
        ],
        out_shape=[
            jax.ShapeDtypeStruct((C, HW), jnp.bfloat16),
            jax.ShapeDtypeStruct((C, GRID1, SUB, NW), _F32),
        ],
    )(x2d, jnp.eye(C, dtype=_F32), Wq4, Wk4, Wv4,
      Wo4, jnp.asarray(_strip_sum_matrix()))

    pool_m = jnp.asarray(_pool_matrix())
    fc = pl.pallas_call(
        _ctx_kernel,
        out_shape=jax.ShapeDtypeStruct((C, C), _F32),
    )(sums, pool_m, Wf)

    out = pl.pallas_call(
        _fmam_kernel,
        grid=(GRID3,),
        in_specs=[
            pl.BlockSpec((C, PIX_PER_STEP), lambda i: (0, i)),
            pl.BlockSpec((C, PIX_PER_STEP), lambda i: (0, i)),
            pl.BlockSpec((C, C), lambda i: (0, 0)),
            pl.BlockSpec((C, C), lambda i: (0, 0)),
            pl.BlockSpec((C, 1), lambda i: (0, 0)),
            pl.BlockSpec((C, 2), lambda i: (0, 0)),
        ],
        out_specs=pl.BlockSpec((C, PIX_PER_STEP), lambda i: (0, i)),
        out_shape=jax.ShapeDtypeStruct((C, HW), _F32),
    )(x2d, spa2d, fc, Wquer, bquer.reshape(C, 1), Wdw)

    return out.reshape(1, C, H, W)
